# Initial kernel scaffold; baseline (speedup 1.0000x reference)
import jax
import jax.numpy as jnp
from jax.experimental import pallas as pl


def kernel(x, W1, b1, W2, b2, W3, b3):
    raise NotImplementedError("write your pallas kernel here")


# speedup vs baseline: 1.3435x; 1.3435x over previous
import jax
import jax.numpy as jnp
from jax.experimental import pallas as pl
from jax.experimental.pallas import tpu as pltpu


def _fused_kernel(x_ref, w1_ref, b1_ref, w2_ref, b2_ref, w3_ref, b3_ref,
                  out_ref, xpad_ref, h1pad_ref):
    T = x_ref.shape[1]
    K1 = w1_ref.shape[0]
    K2 = w2_ref.shape[0]
    P1 = K1 // 2
    P2 = K2 // 2

    @pl.when(pl.program_id(0) == 0)
    def _():
        xpad_ref[0:P1, :] = jnp.zeros_like(xpad_ref[0:P1, :])
        xpad_ref[P1 + T:, :] = jnp.zeros_like(xpad_ref[P1 + T:, :])
        h1pad_ref[0:P2, :] = jnp.zeros_like(h1pad_ref[0:P2, :])
        h1pad_ref[P2 + T:, :] = jnp.zeros_like(h1pad_ref[P2 + T:, :])

    xpad_ref[P1:P1 + T, :] = x_ref[0]

    acc = None
    for k in range(K1):
        part = jnp.dot(xpad_ref[k:k + T, :], w1_ref[k],
                       preferred_element_type=jnp.float32)
        acc = part if acc is None else acc + part
    h1pad_ref[P2:P2 + T, :] = jnp.maximum(acc + b1_ref[:], 0.0)

    acc = None
    for k in range(K2):
        part = jnp.dot(h1pad_ref[k:k + T, :], w2_ref[k],
                       preferred_element_type=jnp.float32)
        acc = part if acc is None else acc + part
    h2 = jnp.maximum(acc + b2_ref[:], 0.0)

    out_ref[0] = jnp.dot(h2, w3_ref[:], preferred_element_type=jnp.float32) \
        + b3_ref[:]


def kernel(x, W1, b1, W2, b2, W3, b3):
    B, T, C = x.shape
    H, _, K1 = W1.shape
    _, _, K2 = W2.shape
    O = W3.shape[0]

    W1t = jnp.transpose(W1, (2, 1, 0))
    W2t = jnp.transpose(W2, (2, 1, 0))
    W3t = jnp.transpose(W3[:, :, 0], (1, 0))

    out = pl.pallas_call(
        _fused_kernel,
        grid=(B,),
        in_specs=[
            pl.BlockSpec((1, T, C), lambda b: (b, 0, 0)),
            pl.BlockSpec((K1, C, H), lambda b: (0, 0, 0)),
            pl.BlockSpec((1, H), lambda b: (0, 0)),
            pl.BlockSpec((K2, H, H), lambda b: (0, 0, 0)),
            pl.BlockSpec((1, H), lambda b: (0, 0)),
            pl.BlockSpec((H, O), lambda b: (0, 0)),
            pl.BlockSpec((1, O), lambda b: (0, 0)),
        ],
        out_specs=pl.BlockSpec((1, T, O), lambda b: (b, 0, 0)),
        out_shape=jax.ShapeDtypeStruct((B, T, O), jnp.float32),
        scratch_shapes=[
            pltpu.VMEM((T + 2 * (K1 // 2), C), jnp.float32),
            pltpu.VMEM((T + 2 * (K2 // 2), H), jnp.float32),
        ],
    )(x, W1t, b1[None, :], W2t, b2[None, :], W3t, b3[None, :])
    return out
